# baseline (speedup 1.0000x reference)
import functools

import jax
import jax.numpy as jnp
from jax.experimental import pallas as pl
from jax.experimental.pallas import tpu as pltpu

_RS = 256
_ST = 56


def _fused_kernel(x_ref, w1_ref, w2_ref, r1_ref, r1b_ref,
                  c2_ref, *, W, H, st):
    s = pl.program_id(1)
    L = st * _RS
    S = (st + 2) * _RS + 128
    base = pl.multiple_of(s * L, 128)

    slabs = [x_ref[0, :, pl.ds(base + ky * _RS, S + 128)] for ky in range(3)]
    slab = jnp.concatenate(
        [slabs[ky][:, kx:kx + S]
         for ky in range(3) for kx in range(3)], axis=0)
    r1 = jnp.dot(w1_ref[...], slab, preferred_element_type=jnp.float32)
    r1 = jnp.maximum(r1, 0.0)

    r1s = (r1[:, _RS:(st + 1) * _RS]
           .reshape(64, st, _RS)[:, :, :W])
    r1_ref[0] = r1s
    r1b_ref[0] = r1s

    lane = jax.lax.broadcasted_iota(jnp.int32, (1, S), 1)
    b = lane & (_RS - 1)
    sa = s * st + (lane >> 8)
    valid = (b < W) & (sa >= 1) & (sa <= H)
    r1b = jnp.where(valid, r1.astype(jnp.bfloat16), jnp.bfloat16(0.0))

    def piece(ky, kx):
        off = ky * _RS + kx - 1
        if off < 0:
            return jnp.pad(r1b[:, :L - 1], ((0, 0), (1, 0)))
        return r1b[:, off:off + L]
    tap = jnp.concatenate(
        [piece(ky, kx) for kx in range(3) for ky in range(3)]
        + [slab[32:40, _RS:_RS + L]], axis=0)
    out2 = jnp.dot(w2_ref[...], tap,
                   preferred_element_type=jnp.float32)
    c2_ref[0] = out2.reshape(64, st, _RS)[:, :, :W]


def kernel(x, w1, b1, w2, b2):
    N, Cin, H, W = x.shape
    Cout = w1.shape[0]
    st = _ST if H % _ST == 0 else 8
    nstrip = H // st
    LT = (H + 7) * _RS

    xb = jnp.concatenate([x, jnp.ones((N, 1, H, W), x.dtype)], axis=1)
    xr = jnp.pad(xb, ((0, 0), (0, 8 - Cin - 1), (2, 5), (1, _RS - W - 1)))
    xr = xr.astype(jnp.bfloat16).reshape(N, 8, LT)

    w1e = jnp.transpose(w1, (0, 2, 3, 1)).reshape(Cout, 9, Cin)
    w1t = jnp.pad(w1e, ((0, 0), (0, 0), (0, 8 - Cin))).reshape(Cout, 9, 8)
    w1t = w1t.at[:, 4, Cin].set(b1)
    w1t = w1t.reshape(Cout, 72).astype(jnp.bfloat16)
    w2t4 = jnp.transpose(w2, (2, 3, 0, 1))
    w2cols = [w2t4[ky, kx] for kx in range(3) for ky in range(3)]
    w2bias = jnp.zeros((Cout, 8), w2.dtype).at[:, Cin].set(b2)
    w2p = jnp.concatenate(w2cols + [w2bias], axis=1)
    w2p = w2p.astype(jnp.bfloat16)

    kern = functools.partial(_fused_kernel, W=W, H=H, st=st)
    r1, r1b, c2 = pl.pallas_call(
        kern,
        out_shape=(
            jax.ShapeDtypeStruct((N, Cout, H, W), jnp.float32),
            jax.ShapeDtypeStruct((N, Cout, H, W), jnp.float32),
            jax.ShapeDtypeStruct((N, Cout, H, W), jnp.float32),
        ),
        grid_spec=pltpu.PrefetchScalarGridSpec(
            num_scalar_prefetch=0,
            grid=(N, nstrip),
            in_specs=[
                pl.BlockSpec((1, 8, LT), lambda n, s: (n, 0, 0)),
                pl.BlockSpec((Cout, 72), lambda n, s: (0, 0)),
                pl.BlockSpec((Cout, 9 * Cout + 8), lambda n, s: (0, 0)),
            ],
            out_specs=(
                pl.BlockSpec((1, Cout, st, W), lambda n, s: (n, 0, s, 0)),
                pl.BlockSpec((1, Cout, st, W), lambda n, s: (n, 0, s, 0)),
                pl.BlockSpec((1, Cout, st, W), lambda n, s: (n, 0, s, 0)),
            ),
        ),
        compiler_params=pltpu.CompilerParams(
            dimension_semantics=("parallel", "arbitrary")),
    )(xr, w1t, w2p)

    return [r1, r1b, c2]


# speedup vs baseline: 1.0209x; 1.0209x over previous
import functools

import jax
import jax.numpy as jnp
from jax.experimental import pallas as pl
from jax.experimental.pallas import tpu as pltpu

_RS = 256
_ST = 56


def _fused_kernel(x_ref, w1_ref, w2_ref, r1_ref, r1b_ref,
                  c2_ref, *, W, H, st):
    s = pl.program_id(1)
    L = st * _RS
    S = (st + 2) * _RS + 128
    base = pl.multiple_of(s * L, 128)

    slabs = [x_ref[0, :, pl.ds(base + ky * _RS, S + 128)] for ky in range(3)]
    slab = jnp.concatenate(
        [slabs[ky][:, kx:kx + S]
         for ky in range(3) for kx in range(3)], axis=0)
    r1 = jnp.dot(w1_ref[...], slab, preferred_element_type=jnp.float32)
    r1 = jnp.maximum(r1, 0.0)

    r1s = (r1[:, _RS + 1:(st + 1) * _RS + 1]
           .reshape(64, st, _RS)[:, :, :W])
    r1_ref[0] = r1s
    r1b_ref[0] = r1s

    lane = jax.lax.broadcasted_iota(jnp.int32, (1, S), 1)
    cp = lane & (_RS - 1)
    rp = s * st + (lane >> 8)
    valid = (cp >= 1) & (cp <= W) & (rp >= 1) & (rp <= H)
    r1b = jnp.where(valid, r1.astype(jnp.bfloat16), jnp.bfloat16(0.0))
    tap = jnp.concatenate(
        [r1b[:, ky * _RS + kx:ky * _RS + kx + L]
         for kx in range(3) for ky in range(3)]
        + [slab[32:40, _RS + 1:_RS + 1 + L]], axis=0)
    out2 = jnp.dot(w2_ref[...], tap,
                   preferred_element_type=jnp.float32)
    c2_ref[0] = out2.reshape(64, st, _RS)[:, :, :W]


def kernel(x, w1, b1, w2, b2):
    N, Cin, H, W = x.shape
    Cout = w1.shape[0]
    st = _ST if H % _ST == 0 else 8
    nstrip = H // st
    LT = (H + 7) * _RS

    xb = jnp.concatenate([x, jnp.ones((N, 1, H, W), x.dtype)], axis=1)
    xr = jnp.pad(xb, ((0, 0), (0, 8 - Cin - 1), (2, 5), (2, _RS - W - 2)))
    xr = xr.astype(jnp.bfloat16).reshape(N, 8, LT)

    w1e = jnp.transpose(w1, (0, 2, 3, 1)).reshape(Cout, 9, Cin)
    w1t = jnp.pad(w1e, ((0, 0), (0, 0), (0, 8 - Cin))).reshape(Cout, 9, 8)
    w1t = w1t.at[:, 4, Cin].set(b1)
    w1t = w1t.reshape(Cout, 72).astype(jnp.bfloat16)
    w2t4 = jnp.transpose(w2, (2, 3, 0, 1))
    w2cols = [w2t4[ky, kx] for kx in range(3) for ky in range(3)]
    w2bias = jnp.zeros((Cout, 8), w2.dtype).at[:, Cin].set(b2)
    w2p = jnp.concatenate(w2cols + [w2bias], axis=1)
    w2p = w2p.astype(jnp.bfloat16)

    kern = functools.partial(_fused_kernel, W=W, H=H, st=st)
    r1, r1b, c2 = pl.pallas_call(
        kern,
        out_shape=(
            jax.ShapeDtypeStruct((N, Cout, H, W), jnp.float32),
            jax.ShapeDtypeStruct((N, Cout, H, W), jnp.float32),
            jax.ShapeDtypeStruct((N, Cout, H, W), jnp.float32),
        ),
        grid_spec=pltpu.PrefetchScalarGridSpec(
            num_scalar_prefetch=0,
            grid=(N, nstrip),
            in_specs=[
                pl.BlockSpec((1, 8, LT), lambda n, s: (n, 0, 0)),
                pl.BlockSpec((Cout, 72), lambda n, s: (0, 0)),
                pl.BlockSpec((Cout, 9 * Cout + 8), lambda n, s: (0, 0)),
            ],
            out_specs=(
                pl.BlockSpec((1, Cout, st, W), lambda n, s: (n, 0, s, 0)),
                pl.BlockSpec((1, Cout, st, W), lambda n, s: (n, 0, s, 0)),
                pl.BlockSpec((1, Cout, st, W), lambda n, s: (n, 0, s, 0)),
            ),
        ),
        compiler_params=pltpu.CompilerParams(
            dimension_semantics=("parallel", "parallel")),
    )(xr, w1t, w2p)

    return [r1, r1b, c2]


# speedup vs baseline: 1.0321x; 1.0109x over previous
import functools

import jax
import jax.numpy as jnp
from jax.experimental import pallas as pl
from jax.experimental.pallas import tpu as pltpu

_RS = 256
_ST = 56


def _fused_kernel(x_ref, w1_ref, w2_ref, r1_ref, r1b_ref,
                  c2_ref, *, W, H, st):
    s = pl.program_id(1)
    L = st * _RS
    S = (st + 2) * _RS + 128
    base = pl.multiple_of(s * L, 128)

    slabs = [x_ref[0, :, pl.ds(base + ky * _RS, S + 128)] for ky in range(3)]
    slab = jnp.concatenate(
        [slabs[ky][:, kx:kx + S]
         for ky in range(3) for kx in range(3)], axis=0)
    r1 = jnp.dot(w1_ref[...], slab, preferred_element_type=jnp.float32)
    r1 = jnp.maximum(r1, 0.0)

    r1s = (r1[:, _RS + 1:(st + 1) * _RS + 1]
           .reshape(64, st, _RS)[:, :, :W])
    r1_ref[0] = r1s
    r1b_ref[0] = r1s

    lane = jax.lax.broadcasted_iota(jnp.int32, (1, S), 1)
    cp = lane & (_RS - 1)
    rp = s * st + (lane >> 8)
    valid = (cp >= 1) & (cp <= W) & (rp >= 1) & (rp <= H)
    r1b = jnp.where(valid, r1.astype(jnp.bfloat16), jnp.bfloat16(0.0))
    h2 = st // 2
    L2 = h2 * _RS
    for j in range(2):
        o = j * L2
        tap = jnp.concatenate(
            [r1b[:, o + ky * _RS + kx:o + ky * _RS + kx + L2]
             for kx in range(3) for ky in range(3)]
            + [slab[32:40, o + _RS + 1:o + _RS + 1 + L2]], axis=0)
        out2 = jnp.dot(w2_ref[...], tap,
                       preferred_element_type=jnp.float32)
        c2_ref[0, :, j * h2:(j + 1) * h2, :] = (
            out2.reshape(64, h2, _RS)[:, :, :W])


def kernel(x, w1, b1, w2, b2):
    N, Cin, H, W = x.shape
    Cout = w1.shape[0]
    st = _ST if H % _ST == 0 else 8
    nstrip = H // st
    LT = (H + 7) * _RS

    xb = jnp.concatenate([x, jnp.ones((N, 1, H, W), x.dtype)], axis=1)
    xr = jnp.pad(xb, ((0, 0), (0, 8 - Cin - 1), (2, 5), (2, _RS - W - 2)))
    xr = xr.astype(jnp.bfloat16).reshape(N, 8, LT)

    w1e = jnp.transpose(w1, (0, 2, 3, 1)).reshape(Cout, 9, Cin)
    w1t = jnp.pad(w1e, ((0, 0), (0, 0), (0, 8 - Cin))).reshape(Cout, 9, 8)
    w1t = w1t.at[:, 4, Cin].set(b1)
    w1t = w1t.reshape(Cout, 72).astype(jnp.bfloat16)
    w2t4 = jnp.transpose(w2, (2, 3, 0, 1))
    w2cols = [w2t4[ky, kx] for kx in range(3) for ky in range(3)]
    w2bias = jnp.zeros((Cout, 8), w2.dtype).at[:, Cin].set(b2)
    w2p = jnp.concatenate(w2cols + [w2bias], axis=1)
    w2p = w2p.astype(jnp.bfloat16)

    kern = functools.partial(_fused_kernel, W=W, H=H, st=st)
    r1, r1b, c2 = pl.pallas_call(
        kern,
        out_shape=(
            jax.ShapeDtypeStruct((N, Cout, H, W), jnp.float32),
            jax.ShapeDtypeStruct((N, Cout, H, W), jnp.float32),
            jax.ShapeDtypeStruct((N, Cout, H, W), jnp.float32),
        ),
        grid_spec=pltpu.PrefetchScalarGridSpec(
            num_scalar_prefetch=0,
            grid=(N, nstrip),
            in_specs=[
                pl.BlockSpec((1, 8, LT), lambda n, s: (n, 0, 0)),
                pl.BlockSpec((Cout, 72), lambda n, s: (0, 0)),
                pl.BlockSpec((Cout, 9 * Cout + 8), lambda n, s: (0, 0)),
            ],
            out_specs=(
                pl.BlockSpec((1, Cout, st, W), lambda n, s: (n, 0, s, 0)),
                pl.BlockSpec((1, Cout, st, W), lambda n, s: (n, 0, s, 0)),
                pl.BlockSpec((1, Cout, st, W), lambda n, s: (n, 0, s, 0)),
            ),
        ),
        compiler_params=pltpu.CompilerParams(
            dimension_semantics=("parallel", "parallel")),
    )(xr, w1t, w2p)

    return [r1, r1b, c2]
